# baseline (speedup 1.0000x reference)
import functools

import jax
import jax.numpy as jnp
from jax.experimental import pallas as pl
from jax.experimental.pallas import tpu as pltpu

NE = 8
TOPK = 2
C = 768
H = 3072
BLK = 256
NA = 2048 * TOPK
NB = NA // BLK + NE - 1
NPAD = NB * BLK
NH = 4
HBLK = H // NH
CHUNK = 256


def _gate_body(x_ref, gw_ref, gb_ref, e_ref, w_ref, r_ref, cnt_ref):
    x = x_ref[...]
    logits = jax.lax.dot_general(
        x, gw_ref[...], (((1,), (1,)), ((), ())),
        preferred_element_type=jnp.float32)
    logits = logits + gb_ref[...]
    m = jnp.max(logits, axis=-1, keepdims=True)
    ex = jnp.exp(logits - m)
    p = ex / jnp.sum(ex, axis=-1, keepdims=True)
    iota = jax.lax.broadcasted_iota(jnp.int32, p.shape, 1)
    m0 = jnp.max(p, axis=-1, keepdims=True)
    i0 = jnp.min(jnp.where(p == m0, iota, NE), axis=-1, keepdims=True)
    p2 = jnp.where(iota == i0, -jnp.inf, p)
    m1 = jnp.max(p2, axis=-1, keepdims=True)
    i1 = jnp.min(jnp.where(p2 == m1, iota, NE), axis=-1, keepdims=True)
    s = m0 + m1
    e_ref[...] = jnp.concatenate([i0, i1], axis=1)
    w_ref[...] = jnp.concatenate([m0 / s, m1 / s], axis=1)

    o0 = (i0 == iota).astype(jnp.float32)
    o1 = (i1 == iota).astype(jnp.float32)
    ssum = o0 + o1
    T = ssum.shape[0]
    ri = jax.lax.broadcasted_iota(jnp.int32, (CHUNK, CHUNK), 0)
    ci = jax.lax.broadcasted_iota(jnp.int32, (CHUNK, CHUNK), 1)
    tril = (ri > ci).astype(jnp.float32)
    carry = jnp.zeros((1, NE), jnp.float32)
    chunks = []
    for k in range(T // CHUNK):
        sc = jax.lax.slice(ssum, (k * CHUNK, 0), ((k + 1) * CHUNK, NE))
        pc = jax.lax.dot_general(
            tril, sc, (((1,), (0,)), ((), ())),
            preferred_element_type=jnp.float32) + carry
        chunks.append(pc)
        carry = carry + jnp.sum(sc, axis=0, keepdims=True)
    prior = jnp.concatenate(chunks, axis=0)
    r0 = jnp.sum(prior * o0, axis=1, keepdims=True)
    r1 = jnp.sum((prior + o0) * o1, axis=1, keepdims=True)
    r_ref[...] = jnp.concatenate([r0, r1], axis=1).astype(jnp.int32)
    cnt_ref[...] = carry.astype(jnp.int32)


def _gelu(h):
    return h * 0.5 * (1.0 + jax.lax.erf(h * 0.7071067811865476))


def _ffn_body(be_ref, valid_ref, xs_ref, w1_ref, b1_ref, w2_ref, b2_ref,
              ys_ref, acc_ref):
    h = pl.program_id(0)
    j = pl.program_id(1)

    @pl.when(valid_ref[j] != 0)
    def _():
        xs = xs_ref[...]
        hid = jax.lax.dot_general(
            xs, w1_ref[0], (((1,), (1,)), ((), ())),
            preferred_element_type=jnp.float32)
        hid = _gelu(hid + b1_ref[0])
        y = jax.lax.dot_general(
            hid, w2_ref[0], (((1,), (1,)), ((), ())),
            preferred_element_type=jnp.float32)

        @pl.when(h == 0)
        def _():
            acc_ref[pl.ds(j * BLK, BLK), :] = y + b2_ref[0]

        @pl.when(h != 0)
        def _():
            acc_ref[pl.ds(j * BLK, BLK), :] += y

        @pl.when(h == NH - 1)
        def _():
            ys_ref[...] = acc_ref[pl.ds(j * BLK, BLK), :]


def kernel(x, gate_w, gate_b, w1, b1, w2, b2):
    Bs, T, _ = x.shape
    xr = x.reshape(Bs * T, C)

    e_idx, wts, rank, counts = pl.pallas_call(
        _gate_body,
        out_shape=(
            jax.ShapeDtypeStruct((Bs * T, TOPK), jnp.int32),
            jax.ShapeDtypeStruct((Bs * T, TOPK), jnp.float32),
            jax.ShapeDtypeStruct((Bs * T, TOPK), jnp.int32),
            jax.ShapeDtypeStruct((1, NE), jnp.int32),
        ),
    )(xr, gate_w, gate_b.reshape(1, NE))

    g = counts[0]
    nb = (g + BLK - 1) // BLK
    startpad = (jnp.cumsum(nb) - nb) * BLK
    cnb = jnp.cumsum(nb)
    total = cnb[-1]
    jidx = jnp.arange(NB, dtype=jnp.int32)
    be_raw = jnp.sum((jidx[:, None] >= cnb[None, :]).astype(jnp.int32), axis=1)
    valid = (jidx < total).astype(jnp.int32)
    be_last = jnp.clip(be_raw, 0, NE - 1)[total - 1]
    be = jnp.where(valid == 1, be_raw, be_last).astype(jnp.int32)

    pos = startpad[e_idx] + rank
    tok = jnp.arange(Bs * T, dtype=jnp.int32)
    sorted_tok = jnp.zeros((NPAD,), jnp.int32)
    sorted_tok = sorted_tok.at[pos[:, 0]].set(tok).at[pos[:, 1]].set(tok)

    xs = jnp.take(xr, sorted_tok, axis=0)

    ys = pl.pallas_call(
        _ffn_body,
        grid_spec=pltpu.PrefetchScalarGridSpec(
            num_scalar_prefetch=2,
            grid=(NH, NB),
            in_specs=[
                pl.BlockSpec((BLK, C), lambda h, j, be, vd: (j, 0)),
                pl.BlockSpec((1, HBLK, C), lambda h, j, be, vd: (be[j], h, 0)),
                pl.BlockSpec((1, 1, HBLK), lambda h, j, be, vd: (be[j], 0, h)),
                pl.BlockSpec((1, C, HBLK), lambda h, j, be, vd: (be[j], 0, h)),
                pl.BlockSpec((1, 1, C), lambda h, j, be, vd: (be[j], 0, 0)),
            ],
            out_specs=pl.BlockSpec(
                (BLK, C),
                lambda h, j, be, vd: (jnp.where(h == NH - 1, j, NB), 0)),
            scratch_shapes=[pltpu.VMEM((NPAD, C), jnp.float32)],
        ),
        out_shape=jax.ShapeDtypeStruct(((NB + 1) * BLK, C), jnp.float32),
        compiler_params=pltpu.CompilerParams(
            dimension_semantics=("arbitrary", "arbitrary"),
        ),
    )(be, valid, xs, w1, b1.reshape(NE, 1, H), w2, b2.reshape(NE, 1, C))

    out = (wts[:, 0:1] * jnp.take(ys, pos[:, 0], axis=0)
           + wts[:, 1:2] * jnp.take(ys, pos[:, 1], axis=0))
    return out.reshape(Bs, T, C)


# speedup vs baseline: 1.7110x; 1.7110x over previous
import functools

import jax
import jax.numpy as jnp
from jax import lax
from jax.experimental import pallas as pl
from jax.experimental.pallas import tpu as pltpu
from jax.experimental.pallas import tpu_sc as plsc

NE = 8
TOPK = 2
C = 768
H = 3072
BLK = 256
T = 2048
NA = T * TOPK
NB = NA // BLK + NE - 1
NPAD = NB * BLK
CHUNK = 256
NWORK = 32
TPW = T // NWORK
LANES = 16


def _gate_body(x_ref, gw_ref, gb_ref, e_ref, w_ref, r_ref, cnt_ref):
    x = x_ref[...]
    logits = jax.lax.dot_general(
        x, gw_ref[...], (((1,), (1,)), ((), ())),
        preferred_element_type=jnp.float32)
    logits = logits + gb_ref[...]
    m = jnp.max(logits, axis=-1, keepdims=True)
    ex = jnp.exp(logits - m)
    p = ex / jnp.sum(ex, axis=-1, keepdims=True)
    iota = jax.lax.broadcasted_iota(jnp.int32, p.shape, 1)
    m0 = jnp.max(p, axis=-1, keepdims=True)
    i0 = jnp.min(jnp.where(p == m0, iota, NE), axis=-1, keepdims=True)
    p2 = jnp.where(iota == i0, -jnp.inf, p)
    m1 = jnp.max(p2, axis=-1, keepdims=True)
    i1 = jnp.min(jnp.where(p2 == m1, iota, NE), axis=-1, keepdims=True)
    s = m0 + m1
    e_ref[...] = jnp.concatenate([i0, i1], axis=1)
    w_ref[...] = jnp.concatenate([m0 / s, m1 / s], axis=1)

    o0 = (i0 == iota).astype(jnp.float32)
    o1 = (i1 == iota).astype(jnp.float32)
    ssum = o0 + o1
    ri = jax.lax.broadcasted_iota(jnp.int32, (CHUNK, CHUNK), 0)
    ci = jax.lax.broadcasted_iota(jnp.int32, (CHUNK, CHUNK), 1)
    tril = (ri > ci).astype(jnp.float32)
    carry = jnp.zeros((1, NE), jnp.float32)
    chunks = []
    for k in range(T // CHUNK):
        sc = jax.lax.slice(ssum, (k * CHUNK, 0), ((k + 1) * CHUNK, NE))
        pc = jax.lax.dot_general(
            tril, sc, (((1,), (0,)), ((), ())),
            preferred_element_type=jnp.float32) + carry
        chunks.append(pc)
        carry = carry + jnp.sum(sc, axis=0, keepdims=True)
    prior = jnp.concatenate(chunks, axis=0)
    r0 = jnp.sum(prior * o0, axis=1, keepdims=True)
    r1 = jnp.sum((prior + o0) * o1, axis=1, keepdims=True)
    r_ref[...] = jnp.concatenate([r0, r1], axis=1).astype(jnp.int32)
    cnt_ref[...] = carry.astype(jnp.int32)


def _gelu(h):
    return h * 0.5 * (1.0 + jax.lax.erf(h * 0.7071067811865476))


def _ffn_body(be_ref, valid_ref, xs_ref, w1_ref, b1_ref, w2_ref, b2_ref,
              wgt_ref, ys_ref):
    j = pl.program_id(0)

    @pl.when(valid_ref[j] != 0)
    def _():
        xs = xs_ref[...]
        hid = jax.lax.dot_general(
            xs, w1_ref[0], (((1,), (1,)), ((), ())),
            preferred_element_type=jnp.float32)
        hid = _gelu(hid + b1_ref[0])
        y = jax.lax.dot_general(
            hid, w2_ref[0], (((1,), (1,)), ((), ())),
            preferred_element_type=jnp.float32)
        ys_ref[...] = (y + b2_ref[0]) * wgt_ref[:, :1]


def _make_dispatch():
    mesh = plsc.VectorSubcoreMesh(core_axis_name="c", subcore_axis_name="s")

    @functools.partial(
        pl.kernel, mesh=mesh,
        out_type=(
            jax.ShapeDtypeStruct((NPAD, C), jnp.float32),
            jax.ShapeDtypeStruct((NPAD, 128), jnp.float32),
        ),
        scratch_types=[
            pltpu.VMEM((TPW, C), jnp.float32),
            pltpu.VMEM((TPW,), jnp.int32),
            pltpu.VMEM((TPW,), jnp.int32),
            pltpu.VMEM((TPW, 128), jnp.float32),
            pltpu.VMEM((TPW, 128), jnp.float32),
            pltpu.SemaphoreType.DMA,
        ],
    )
    def dispatch(x_hbm, pos0_hbm, pos1_hbm, w0_hbm, w1_hbm, xs_hbm, wgt_hbm,
                 rows_v, idx0_v, idx1_v, wv0_v, wv1_v, sem):
        wid = lax.axis_index("s") * 2 + lax.axis_index("c")
        base = wid * TPW
        pltpu.sync_copy(x_hbm.at[pl.ds(base, TPW)], rows_v)
        pltpu.sync_copy(pos0_hbm.at[pl.ds(base, TPW)], idx0_v)
        pltpu.sync_copy(pos1_hbm.at[pl.ds(base, TPW)], idx1_v)
        pltpu.sync_copy(w0_hbm.at[pl.ds(base, TPW)], wv0_v)
        pltpu.sync_copy(w1_hbm.at[pl.ds(base, TPW)], wv1_v)
        cp0 = pltpu.async_copy(rows_v, xs_hbm.at[idx0_v], sem)
        cp1 = pltpu.async_copy(rows_v, xs_hbm.at[idx1_v], sem)
        cp2 = pltpu.async_copy(wv0_v, wgt_hbm.at[idx0_v], sem)
        cp3 = pltpu.async_copy(wv1_v, wgt_hbm.at[idx1_v], sem)
        cp0.wait()
        cp1.wait()
        cp2.wait()
        cp3.wait()

    return dispatch


def _make_combine():
    mesh = plsc.VectorSubcoreMesh(core_axis_name="c", subcore_axis_name="s")

    @functools.partial(
        pl.kernel, mesh=mesh,
        out_type=jax.ShapeDtypeStruct((T, C), jnp.float32),
        scratch_types=[
            pltpu.VMEM((TPW, C), jnp.float32),
            pltpu.VMEM((TPW, C), jnp.float32),
            pltpu.VMEM((TPW,), jnp.int32),
            pltpu.VMEM((TPW,), jnp.int32),
            pltpu.SemaphoreType.DMA,
        ],
    )
    def combine(ys_hbm, pos0_hbm, pos1_hbm, out_hbm,
                buf0_v, buf1_v, idx0_v, idx1_v, sem):
        wid = lax.axis_index("s") * 2 + lax.axis_index("c")
        base = wid * TPW
        pltpu.sync_copy(pos0_hbm.at[pl.ds(base, TPW)], idx0_v)
        pltpu.sync_copy(pos1_hbm.at[pl.ds(base, TPW)], idx1_v)
        cp0 = pltpu.async_copy(ys_hbm.at[idx0_v], buf0_v, sem)
        cp1 = pltpu.async_copy(ys_hbm.at[idx1_v], buf1_v, sem)
        cp0.wait()
        cp1.wait()

        def token_body(i, _):
            for cchunk in range(C // LANES):
                sl = pl.ds(cchunk * LANES, LANES)
                buf0_v[i, sl] = buf0_v[i, sl] + buf1_v[i, sl]
            return 0

        lax.fori_loop(0, TPW, token_body, 0)
        pltpu.sync_copy(buf0_v, out_hbm.at[pl.ds(base, TPW)])

    return combine


def kernel(x, gate_w, gate_b, w1, b1, w2, b2):
    Bs = x.shape[0]
    xr = x.reshape(Bs * T, C)

    e_idx, wts, rank, counts = pl.pallas_call(
        _gate_body,
        out_shape=(
            jax.ShapeDtypeStruct((T, TOPK), jnp.int32),
            jax.ShapeDtypeStruct((T, TOPK), jnp.float32),
            jax.ShapeDtypeStruct((T, TOPK), jnp.int32),
            jax.ShapeDtypeStruct((1, NE), jnp.int32),
        ),
    )(xr, gate_w, gate_b.reshape(1, NE))

    g = counts[0]
    nb = (g + BLK - 1) // BLK
    startpad = (jnp.cumsum(nb) - nb) * BLK
    cnb = jnp.cumsum(nb)
    total = cnb[-1]
    jidx = jnp.arange(NB, dtype=jnp.int32)
    be_raw = jnp.sum((jidx[:, None] >= cnb[None, :]).astype(jnp.int32), axis=1)
    valid = (jidx < total).astype(jnp.int32)
    be_last = jnp.clip(be_raw, 0, NE - 1)[total - 1]
    be = jnp.where(valid == 1, be_raw, be_last).astype(jnp.int32)

    pos = startpad[e_idx] + rank
    pos0 = pos[:, 0]
    pos1 = pos[:, 1]

    wb0 = jnp.broadcast_to(wts[:, 0:1], (T, 128))
    wb1 = jnp.broadcast_to(wts[:, 1:2], (T, 128))
    xs, wgtpad = _make_dispatch()(xr, pos0, pos1, wb0, wb1)

    ys = pl.pallas_call(
        _ffn_body,
        grid_spec=pltpu.PrefetchScalarGridSpec(
            num_scalar_prefetch=2,
            grid=(NB,),
            in_specs=[
                pl.BlockSpec((BLK, C), lambda j, be, vd: (j, 0)),
                pl.BlockSpec((1, H, C), lambda j, be, vd: (be[j], 0, 0)),
                pl.BlockSpec((1, 1, H), lambda j, be, vd: (be[j], 0, 0)),
                pl.BlockSpec((1, C, H), lambda j, be, vd: (be[j], 0, 0)),
                pl.BlockSpec((1, 1, C), lambda j, be, vd: (be[j], 0, 0)),
                pl.BlockSpec((BLK, 128), lambda j, be, vd: (j, 0)),
            ],
            out_specs=pl.BlockSpec((BLK, C), lambda j, be, vd: (j, 0)),
        ),
        out_shape=jax.ShapeDtypeStruct((NPAD, C), jnp.float32),
        compiler_params=pltpu.CompilerParams(
            dimension_semantics=("arbitrary",),
        ),
    )(be, valid, xs, w1, b1.reshape(NE, 1, H), w2, b2.reshape(NE, 1, C),
      wgtpad)

    out = _make_combine()(ys, pos0, pos1)
    return out.reshape(Bs, T, C)


# speedup vs baseline: 1.8156x; 1.0612x over previous
import functools

import jax
import jax.numpy as jnp
from jax import lax
from jax.experimental import pallas as pl
from jax.experimental.pallas import tpu as pltpu
from jax.experimental.pallas import tpu_sc as plsc

NE = 8
TOPK = 2
C = 768
H = 3072
BLK = 256
T = 2048
NA = T * TOPK
NB = NA // BLK + NE - 1
NPAD = NB * BLK
CHUNK = 256
NWORK = 32
TPW = T // NWORK
LANES = 16


def _gate_body(x_ref, gw_ref, gb_ref, p_ref, w_ref, cnt_ref):
    x = x_ref[...]
    logits = jax.lax.dot_general(
        x, gw_ref[...], (((1,), (1,)), ((), ())),
        preferred_element_type=jnp.float32)
    logits = logits + gb_ref[...]
    m = jnp.max(logits, axis=-1, keepdims=True)
    ex = jnp.exp(logits - m)
    p = ex / jnp.sum(ex, axis=-1, keepdims=True)
    iota = jax.lax.broadcasted_iota(jnp.int32, p.shape, 1)
    m0 = jnp.max(p, axis=-1, keepdims=True)
    i0 = jnp.min(jnp.where(p == m0, iota, NE), axis=-1, keepdims=True)
    p2 = jnp.where(iota == i0, -jnp.inf, p)
    m1 = jnp.max(p2, axis=-1, keepdims=True)
    i1 = jnp.min(jnp.where(p2 == m1, iota, NE), axis=-1, keepdims=True)
    s = m0 + m1
    w_ref[...] = jnp.concatenate([m0 / s, m1 / s], axis=1)

    o0 = (i0 == iota).astype(jnp.float32)
    o1 = (i1 == iota).astype(jnp.float32)
    ssum = o0 + o1
    ri = jax.lax.broadcasted_iota(jnp.int32, (CHUNK, CHUNK), 0)
    ci = jax.lax.broadcasted_iota(jnp.int32, (CHUNK, CHUNK), 1)
    tril = (ri > ci).astype(jnp.float32)
    carry = jnp.zeros((1, NE), jnp.float32)
    chunks = []
    for k in range(T // CHUNK):
        sc = jax.lax.slice(ssum, (k * CHUNK, 0), ((k + 1) * CHUNK, NE))
        pc = jax.lax.dot_general(
            tril, sc, (((1,), (0,)), ((), ())),
            preferred_element_type=jnp.float32) + carry
        chunks.append(pc)
        carry = carry + jnp.sum(sc, axis=0, keepdims=True)
    prior = jnp.concatenate(chunks, axis=0)
    r0 = jnp.sum(prior * o0, axis=1, keepdims=True)
    r1 = jnp.sum((prior + o0) * o1, axis=1, keepdims=True)
    nbf = jnp.ceil(carry * (1.0 / BLK))
    tri = (jax.lax.broadcasted_iota(jnp.int32, (NE, NE), 0)
           <= jax.lax.broadcasted_iota(jnp.int32, (NE, NE), 1))
    cnbf = jax.lax.dot_general(
        nbf, tri.astype(jnp.float32), (((1,), (0,)), ((), ())),
        preferred_element_type=jnp.float32)
    startpad = (cnbf - nbf) * BLK
    p0 = r0 + jnp.sum(o0 * startpad, axis=1, keepdims=True)
    p1 = r1 + jnp.sum(o1 * startpad, axis=1, keepdims=True)
    p_ref[...] = jnp.concatenate([p0, p1], axis=1).astype(jnp.int32)
    cnt_ref[...] = carry.astype(jnp.int32)


def _gelu(h):
    return h * 0.5 * (1.0 + jax.lax.erf(h * 0.7071067811865476))


def _ffn_body(be_ref, valid_ref, xs_ref, w1_ref, b1_ref, w2_ref, b2_ref,
              wgt_ref, ys_ref):
    j = pl.program_id(0)

    @pl.when(valid_ref[j] != 0)
    def _():
        xs = xs_ref[...]
        hid = jax.lax.dot_general(
            xs, w1_ref[0], (((1,), (1,)), ((), ())),
            preferred_element_type=jnp.float32)
        hid = _gelu(hid + b1_ref[0])
        y = jax.lax.dot_general(
            hid, w2_ref[0], (((1,), (1,)), ((), ())),
            preferred_element_type=jnp.float32)
        ys_ref[...] = (y + b2_ref[0]) * wgt_ref[:, :1]


def _make_dispatch():
    mesh = plsc.VectorSubcoreMesh(core_axis_name="c", subcore_axis_name="s")

    @functools.partial(
        pl.kernel, mesh=mesh,
        out_type=(
            jax.ShapeDtypeStruct((NPAD, C), jnp.float32),
            jax.ShapeDtypeStruct((NPAD, 128), jnp.float32),
        ),
        scratch_types=[
            pltpu.VMEM((TPW, C), jnp.float32),
            pltpu.VMEM((TPW,), jnp.int32),
            pltpu.VMEM((TPW,), jnp.int32),
            pltpu.VMEM((TPW, 128), jnp.float32),
            pltpu.VMEM((TPW, 128), jnp.float32),
            pltpu.SemaphoreType.DMA,
            pltpu.SemaphoreType.DMA,
        ],
    )
    def dispatch(x_hbm, pos0_hbm, pos1_hbm, w0_hbm, w1_hbm, xs_hbm, wgt_hbm,
                 rows_v, idx0_v, idx1_v, wv0_v, wv1_v, lsem, sem):
        wid = lax.axis_index("s") * 2 + lax.axis_index("c")
        base = wid * TPW
        ld0 = pltpu.async_copy(x_hbm.at[pl.ds(base, TPW)], rows_v, lsem)
        ld1 = pltpu.async_copy(pos0_hbm.at[pl.ds(base, TPW)], idx0_v, lsem)
        ld2 = pltpu.async_copy(pos1_hbm.at[pl.ds(base, TPW)], idx1_v, lsem)
        ld3 = pltpu.async_copy(w0_hbm.at[pl.ds(base, TPW)], wv0_v, lsem)
        ld4 = pltpu.async_copy(w1_hbm.at[pl.ds(base, TPW)], wv1_v, lsem)
        ld0.wait()
        ld1.wait()
        ld2.wait()
        ld3.wait()
        ld4.wait()
        cp0 = pltpu.async_copy(rows_v, xs_hbm.at[idx0_v], sem)
        cp1 = pltpu.async_copy(rows_v, xs_hbm.at[idx1_v], sem)
        cp2 = pltpu.async_copy(wv0_v, wgt_hbm.at[idx0_v], sem)
        cp3 = pltpu.async_copy(wv1_v, wgt_hbm.at[idx1_v], sem)
        cp0.wait()
        cp1.wait()
        cp2.wait()
        cp3.wait()

    return dispatch


def _make_combine():
    mesh = plsc.VectorSubcoreMesh(core_axis_name="c", subcore_axis_name="s")

    @functools.partial(
        pl.kernel, mesh=mesh,
        out_type=jax.ShapeDtypeStruct((T, C), jnp.float32),
        scratch_types=[
            pltpu.VMEM((TPW, C), jnp.float32),
            pltpu.VMEM((TPW, C), jnp.float32),
            pltpu.VMEM((TPW,), jnp.int32),
            pltpu.VMEM((TPW,), jnp.int32),
            pltpu.SemaphoreType.DMA,
        ],
    )
    def combine(ys_hbm, pos0_hbm, pos1_hbm, out_hbm,
                buf0_v, buf1_v, idx0_v, idx1_v, sem):
        wid = lax.axis_index("s") * 2 + lax.axis_index("c")
        base = wid * TPW
        pltpu.sync_copy(pos0_hbm.at[pl.ds(base, TPW)], idx0_v)
        pltpu.sync_copy(pos1_hbm.at[pl.ds(base, TPW)], idx1_v)
        cp0 = pltpu.async_copy(ys_hbm.at[idx0_v], buf0_v, sem)
        cp1 = pltpu.async_copy(ys_hbm.at[idx1_v], buf1_v, sem)
        cp0.wait()
        cp1.wait()

        def token_body(i, _):
            for cchunk in range(C // LANES):
                sl = pl.ds(cchunk * LANES, LANES)
                buf0_v[i, sl] = buf0_v[i, sl] + buf1_v[i, sl]
            return 0

        lax.fori_loop(0, TPW, token_body, 0)
        pltpu.sync_copy(buf0_v, out_hbm.at[pl.ds(base, TPW)])

    return combine


def kernel(x, gate_w, gate_b, w1, b1, w2, b2):
    Bs = x.shape[0]
    xr = x.reshape(Bs * T, C)

    pos, wts, counts = pl.pallas_call(
        _gate_body,
        out_shape=(
            jax.ShapeDtypeStruct((T, TOPK), jnp.int32),
            jax.ShapeDtypeStruct((T, TOPK), jnp.float32),
            jax.ShapeDtypeStruct((1, NE), jnp.int32),
        ),
    )(xr, gate_w, gate_b.reshape(1, NE))

    g = counts[0]
    nb = (g + BLK - 1) // BLK
    cnb = jnp.cumsum(nb)
    total = cnb[-1]
    jidx = jnp.arange(NB, dtype=jnp.int32)
    be_raw = jnp.sum((jidx[:, None] >= cnb[None, :]).astype(jnp.int32), axis=1)
    valid = (jidx < total).astype(jnp.int32)
    be_last = jnp.clip(be_raw, 0, NE - 1)[total - 1]
    be = jnp.where(valid == 1, be_raw, be_last).astype(jnp.int32)

    pos0 = pos[:, 0]
    pos1 = pos[:, 1]

    wb0 = jnp.broadcast_to(wts[:, 0:1], (T, 128))
    wb1 = jnp.broadcast_to(wts[:, 1:2], (T, 128))
    xs, wgtpad = _make_dispatch()(xr, pos0, pos1, wb0, wb1)

    ys = pl.pallas_call(
        _ffn_body,
        grid_spec=pltpu.PrefetchScalarGridSpec(
            num_scalar_prefetch=2,
            grid=(NB,),
            in_specs=[
                pl.BlockSpec((BLK, C), lambda j, be, vd: (j, 0)),
                pl.BlockSpec((1, H, C), lambda j, be, vd: (be[j], 0, 0)),
                pl.BlockSpec((1, 1, H), lambda j, be, vd: (be[j], 0, 0)),
                pl.BlockSpec((1, C, H), lambda j, be, vd: (be[j], 0, 0)),
                pl.BlockSpec((1, 1, C), lambda j, be, vd: (be[j], 0, 0)),
                pl.BlockSpec((BLK, 128), lambda j, be, vd: (j, 0)),
            ],
            out_specs=pl.BlockSpec((BLK, C), lambda j, be, vd: (j, 0)),
        ),
        out_shape=jax.ShapeDtypeStruct((NPAD, C), jnp.float32),
        compiler_params=pltpu.CompilerParams(
            dimension_semantics=("arbitrary",),
        ),
    )(be, valid, xs, w1, b1.reshape(NE, 1, H), w2, b2.reshape(NE, 1, C),
      wgtpad)

    out = _make_combine()(ys, pos0, pos1)
    return out.reshape(Bs, T, C)


# speedup vs baseline: 1.9469x; 1.0723x over previous
import functools

import jax
import jax.numpy as jnp
from jax import lax
from jax.experimental import pallas as pl
from jax.experimental.pallas import tpu as pltpu
from jax.experimental.pallas import tpu_sc as plsc

NE = 8
TOPK = 2
C = 768
H = 3072
BLK = 512
T = 2048
NA = T * TOPK
NB = NA // BLK + NE - 1
NPAD = NB * BLK
CHUNK = 256
NWORK = 32
TPW = T // NWORK
LANES = 16


def _gate_body(x_ref, gw_ref, gb_ref, p_ref, w_ref, cnt_ref):
    x = x_ref[...]
    logits = jax.lax.dot_general(
        x, gw_ref[...], (((1,), (1,)), ((), ())),
        preferred_element_type=jnp.float32)
    logits = logits + gb_ref[...]
    m = jnp.max(logits, axis=-1, keepdims=True)
    ex = jnp.exp(logits - m)
    p = ex / jnp.sum(ex, axis=-1, keepdims=True)
    iota = jax.lax.broadcasted_iota(jnp.int32, p.shape, 1)
    m0 = jnp.max(p, axis=-1, keepdims=True)
    i0 = jnp.min(jnp.where(p == m0, iota, NE), axis=-1, keepdims=True)
    p2 = jnp.where(iota == i0, -jnp.inf, p)
    m1 = jnp.max(p2, axis=-1, keepdims=True)
    i1 = jnp.min(jnp.where(p2 == m1, iota, NE), axis=-1, keepdims=True)
    s = m0 + m1
    w_ref[...] = jnp.concatenate([m0 / s, m1 / s], axis=1)

    o0 = (i0 == iota).astype(jnp.float32)
    o1 = (i1 == iota).astype(jnp.float32)
    ssum = o0 + o1
    ri = jax.lax.broadcasted_iota(jnp.int32, (CHUNK, CHUNK), 0)
    ci = jax.lax.broadcasted_iota(jnp.int32, (CHUNK, CHUNK), 1)
    tril = (ri > ci).astype(jnp.float32)
    carry = jnp.zeros((1, NE), jnp.float32)
    chunks = []
    for k in range(T // CHUNK):
        sc = jax.lax.slice(ssum, (k * CHUNK, 0), ((k + 1) * CHUNK, NE))
        pc = jax.lax.dot_general(
            tril, sc, (((1,), (0,)), ((), ())),
            preferred_element_type=jnp.float32) + carry
        chunks.append(pc)
        carry = carry + jnp.sum(sc, axis=0, keepdims=True)
    prior = jnp.concatenate(chunks, axis=0)
    r0 = jnp.sum(prior * o0, axis=1, keepdims=True)
    r1 = jnp.sum((prior + o0) * o1, axis=1, keepdims=True)
    nbf = jnp.ceil(carry * (1.0 / BLK))
    tri = (jax.lax.broadcasted_iota(jnp.int32, (NE, NE), 0)
           <= jax.lax.broadcasted_iota(jnp.int32, (NE, NE), 1))
    cnbf = jax.lax.dot_general(
        nbf, tri.astype(jnp.float32), (((1,), (0,)), ((), ())),
        preferred_element_type=jnp.float32)
    startpad = (cnbf - nbf) * BLK
    p0 = r0 + jnp.sum(o0 * startpad, axis=1, keepdims=True)
    p1 = r1 + jnp.sum(o1 * startpad, axis=1, keepdims=True)
    p_ref[...] = jnp.concatenate([p0, p1], axis=1).astype(jnp.int32)
    cnt_ref[...] = carry.astype(jnp.int32)


def _gelu(h):
    return h * 0.5 * (1.0 + jax.lax.erf(h * 0.7071067811865476))


def _ffn_body(be_ref, valid_ref, xs_ref, w1_ref, b1_ref, w2_ref, b2_ref,
              wgt_ref, ys_ref):
    j = pl.program_id(0)

    @pl.when(valid_ref[j] != 0)
    def _():
        xs = xs_ref[...]
        hid = jax.lax.dot_general(
            xs, w1_ref[0], (((1,), (1,)), ((), ())),
            preferred_element_type=jnp.float32)
        hid = _gelu(hid + b1_ref[0])
        y = jax.lax.dot_general(
            hid, w2_ref[0], (((1,), (1,)), ((), ())),
            preferred_element_type=jnp.float32)
        ys_ref[...] = (y + b2_ref[0]) * wgt_ref[:, :1]


def _make_dispatch():
    mesh = plsc.VectorSubcoreMesh(core_axis_name="c", subcore_axis_name="s")

    @functools.partial(
        pl.kernel, mesh=mesh,
        out_type=(
            jax.ShapeDtypeStruct((NPAD, C), jnp.float32),
            jax.ShapeDtypeStruct((NPAD, 128), jnp.float32),
        ),
        scratch_types=[
            pltpu.VMEM((TPW, C), jnp.float32),
            pltpu.VMEM((TPW,), jnp.int32),
            pltpu.VMEM((TPW,), jnp.int32),
            pltpu.VMEM((TPW, 128), jnp.float32),
            pltpu.VMEM((TPW, 128), jnp.float32),
            pltpu.SemaphoreType.DMA,
            pltpu.SemaphoreType.DMA,
        ],
    )
    def dispatch(x_hbm, pos0_hbm, pos1_hbm, w0_hbm, w1_hbm, xs_hbm, wgt_hbm,
                 rows_v, idx0_v, idx1_v, wv0_v, wv1_v, lsem, sem):
        wid = lax.axis_index("s") * 2 + lax.axis_index("c")
        base = wid * TPW
        ld0 = pltpu.async_copy(x_hbm.at[pl.ds(base, TPW)], rows_v, lsem)
        ld1 = pltpu.async_copy(pos0_hbm.at[pl.ds(base, TPW)], idx0_v, lsem)
        ld2 = pltpu.async_copy(pos1_hbm.at[pl.ds(base, TPW)], idx1_v, lsem)
        ld3 = pltpu.async_copy(w0_hbm.at[pl.ds(base, TPW)], wv0_v, lsem)
        ld4 = pltpu.async_copy(w1_hbm.at[pl.ds(base, TPW)], wv1_v, lsem)
        ld0.wait()
        ld1.wait()
        ld2.wait()
        ld3.wait()
        ld4.wait()
        cp0 = pltpu.async_copy(rows_v, xs_hbm.at[idx0_v], sem)
        cp1 = pltpu.async_copy(rows_v, xs_hbm.at[idx1_v], sem)
        cp2 = pltpu.async_copy(wv0_v, wgt_hbm.at[idx0_v], sem)
        cp3 = pltpu.async_copy(wv1_v, wgt_hbm.at[idx1_v], sem)
        cp0.wait()
        cp1.wait()
        cp2.wait()
        cp3.wait()

    return dispatch


def _make_combine():
    mesh = plsc.VectorSubcoreMesh(core_axis_name="c", subcore_axis_name="s")

    @functools.partial(
        pl.kernel, mesh=mesh,
        out_type=jax.ShapeDtypeStruct((T, C), jnp.float32),
        scratch_types=[
            pltpu.VMEM((TPW, C), jnp.float32),
            pltpu.VMEM((TPW, C), jnp.float32),
            pltpu.VMEM((TPW,), jnp.int32),
            pltpu.VMEM((TPW,), jnp.int32),
            pltpu.SemaphoreType.DMA,
        ],
    )
    def combine(ys_hbm, pos0_hbm, pos1_hbm, out_hbm,
                buf0_v, buf1_v, idx0_v, idx1_v, sem):
        wid = lax.axis_index("s") * 2 + lax.axis_index("c")
        base = wid * TPW
        pltpu.sync_copy(pos0_hbm.at[pl.ds(base, TPW)], idx0_v)
        pltpu.sync_copy(pos1_hbm.at[pl.ds(base, TPW)], idx1_v)
        cp0 = pltpu.async_copy(ys_hbm.at[idx0_v], buf0_v, sem)
        cp1 = pltpu.async_copy(ys_hbm.at[idx1_v], buf1_v, sem)
        cp0.wait()
        cp1.wait()

        def token_body(i, _):
            for cchunk in range(C // LANES):
                sl = pl.ds(cchunk * LANES, LANES)
                buf0_v[i, sl] = buf0_v[i, sl] + buf1_v[i, sl]
            return 0

        lax.fori_loop(0, TPW, token_body, 0)
        pltpu.sync_copy(buf0_v, out_hbm.at[pl.ds(base, TPW)])

    return combine


def kernel(x, gate_w, gate_b, w1, b1, w2, b2):
    Bs = x.shape[0]
    xr = x.reshape(Bs * T, C)

    pos, wts, counts = pl.pallas_call(
        _gate_body,
        out_shape=(
            jax.ShapeDtypeStruct((T, TOPK), jnp.int32),
            jax.ShapeDtypeStruct((T, TOPK), jnp.float32),
            jax.ShapeDtypeStruct((1, NE), jnp.int32),
        ),
    )(xr, gate_w, gate_b.reshape(1, NE))

    g = counts[0]
    nb = (g + BLK - 1) // BLK
    cnb = jnp.cumsum(nb)
    total = cnb[-1]
    jidx = jnp.arange(NB, dtype=jnp.int32)
    be_raw = jnp.sum((jidx[:, None] >= cnb[None, :]).astype(jnp.int32), axis=1)
    valid = (jidx < total).astype(jnp.int32)
    be_last = jnp.clip(be_raw, 0, NE - 1)[total - 1]
    be = jnp.where(valid == 1, be_raw, be_last).astype(jnp.int32)

    pos0 = pos[:, 0]
    pos1 = pos[:, 1]

    wb0 = jnp.broadcast_to(wts[:, 0:1], (T, 128))
    wb1 = jnp.broadcast_to(wts[:, 1:2], (T, 128))
    xs, wgtpad = _make_dispatch()(xr, pos0, pos1, wb0, wb1)

    ys = pl.pallas_call(
        _ffn_body,
        grid_spec=pltpu.PrefetchScalarGridSpec(
            num_scalar_prefetch=2,
            grid=(NB,),
            in_specs=[
                pl.BlockSpec((BLK, C), lambda j, be, vd: (j, 0)),
                pl.BlockSpec((1, H, C), lambda j, be, vd: (be[j], 0, 0)),
                pl.BlockSpec((1, 1, H), lambda j, be, vd: (be[j], 0, 0)),
                pl.BlockSpec((1, C, H), lambda j, be, vd: (be[j], 0, 0)),
                pl.BlockSpec((1, 1, C), lambda j, be, vd: (be[j], 0, 0)),
                pl.BlockSpec((BLK, 128), lambda j, be, vd: (j, 0)),
            ],
            out_specs=pl.BlockSpec((BLK, C), lambda j, be, vd: (j, 0)),
        ),
        out_shape=jax.ShapeDtypeStruct((NPAD, C), jnp.float32),
        compiler_params=pltpu.CompilerParams(
            dimension_semantics=("arbitrary",),
        ),
    )(be, valid, xs, w1, b1.reshape(NE, 1, H), w2, b2.reshape(NE, 1, C),
      wgtpad)

    out = _make_combine()(ys, pos0, pos1)
    return out.reshape(Bs, T, C)
